# baseline (speedup 1.0000x reference)
import functools
import jax
import jax.numpy as jnp
from jax import lax
from jax.experimental import pallas as pl
from jax.experimental.pallas import tpu as pltpu
from jax.experimental.pallas import tpu_sc as plsc

N = 10000
D = 128
NC = 2
NS = 16
L = 16
B = 64
ND = 10240
SLAB = 8192
SLABP = SLAB + 128


def _iota16():
    return lax.iota(jnp.int32, 16)


def _vmem_scalar(ref, j):
    base = (j // L) * L
    v = ref[pl.ds(base, L)]
    m = _iota16() == (j - base)
    return jnp.max(jnp.where(m, v, jnp.int32(0)))


def _zero_rows(zbuf, dst, row0, nrows):
    h = zbuf.shape[0]
    nfull = nrows // h
    for k in range(nfull):
        pltpu.sync_copy(zbuf, dst.at[pl.ds(row0 + k * h, h)])
    rem = nrows - nfull * h
    if rem > 0:
        pltpu.sync_copy(zbuf.at[pl.ds(0, rem)], dst.at[pl.ds(row0 + nfull * h, rem)])


def _fill_zero(buf):
    rows, cols = buf.shape
    z = jnp.zeros((L,), jnp.float32)

    def row_body(i, _):
        for c in range(cols // L):
            buf[i, pl.ds(c * L, L)] = z
        return 0

    lax.fori_loop(0, rows, row_body, 0)


def _mm_body(a_ref, b_ref, o_ref):
    o_ref[...] = jnp.dot(a_ref[...], b_ref[...], preferred_element_type=jnp.float32)


def _fold_weights(whet_flat, w_mlp1):
    return pl.pallas_call(
        _mm_body,
        out_shape=jax.ShapeDtypeStruct((whet_flat.shape[0], D), jnp.float32),
    )(whet_flat, w_mlp1)


def _pass1_body(x_ref, w_ref, b_ref, c1_ref, h_ref, g1_ref):
    x = x_ref[...]
    h_ref[...] = jnp.maximum(jnp.dot(x, w_ref[...], preferred_element_type=jnp.float32)
                             + b_ref[...], 0.0)
    g1_ref[...] = jnp.dot(x, c1_ref[...], preferred_element_type=jnp.float32)


def _pass1(xp, w_lin0, b_lin0, c1, bm):
    mp = xp.shape[0]
    return pl.pallas_call(
        _pass1_body,
        grid=(mp // bm,),
        in_specs=[
            pl.BlockSpec((bm, D), lambda i: (i, 0)),
            pl.BlockSpec((D, D), lambda i: (0, 0)),
            pl.BlockSpec((1, D), lambda i: (0, 0)),
            pl.BlockSpec((D, D), lambda i: (0, 0)),
        ],
        out_specs=[
            pl.BlockSpec((bm, D), lambda i: (i, 0)),
            pl.BlockSpec((bm, D), lambda i: (i, 0)),
        ],
        out_shape=[
            jax.ShapeDtypeStruct((mp, D), jnp.float32),
            jax.ShapeDtypeStruct((mp, D), jnp.float32),
        ],
    )(xp, w_lin0, b_lin0, c1)


def _tables_body(ss_ref, sn_ref, s7_ref, dv_ref, x4d_ref, rs_ref, rn_ref,
                 c_ref, ts_ref, tn_ref, p7t_ref, dl_ref, *, bn):
    pid = pl.program_id(0)
    rid = pid * bn + lax.broadcasted_iota(jnp.int32, (bn, 1), 0)
    m = (rid < N).astype(jnp.float32)
    rcp_s = rs_ref[...] * m
    rcp_n = rn_ref[...] * m
    dv = dv_ref[...] * m
    x4d = x4d_ref[...] * m
    p5 = sn_ref[...] * rcp_n
    p6 = ss_ref[...] * rcp_s
    p7 = (s7_ref[0] + s7_ref[1]) * rcp_n
    c = c_ref[...]

    def mm(a, w):
        return jnp.dot(a, w, preferred_element_type=jnp.float32)

    ts = mm(dv, c[0, 1]) + mm(p6, c[0, 5])
    tn = mm(dv, c[0, 2]) + mm(p5, c[0, 4])
    p7t = mm(p7, c[0, 6])
    zd = (mm(dv, c[1, 0] + c[1, 1] + c[1, 2]) + mm(x4d, c[1, 3])
          + mm(p5, c[1, 4]) + mm(p6, c[1, 5]) + mm(p7, c[1, 6]))
    base_diag = mm(x4d, c[0, 3]) + mm(dv, c[0, 0]) + ts + tn + p7t
    ts_ref[...] = ts * m
    tn_ref[...] = tn * m
    p7t_ref[...] = p7t * m
    dl_ref[...] = (zd - base_diag) * m


def _tables(s_s, s_n, s7p, dvec, x4d, rcp_s, rcp_n, call):
    bn = 1024
    spec = pl.BlockSpec((bn, D), lambda i: (i, 0))
    out = jax.ShapeDtypeStruct((ND, D), jnp.float32)
    return pl.pallas_call(
        functools.partial(_tables_body, bn=bn),
        grid=(ND // bn,),
        in_specs=[
            spec, spec,
            pl.BlockSpec((2, bn, D), lambda i: (0, i, 0)),
            spec, spec, spec, spec,
            pl.BlockSpec((2, 7, D, D), lambda i: (0, 0, 0, 0)),
        ],
        out_specs=[spec, spec, spec, spec],
        out_shape=[out, out, out, out],
    )(s_s, s_n, s7p, dvec, x4d, rcp_s, rcp_n, call)


def _final_body(x4_ref, g1_ref, gs_ref, c4_ref, b_ref, o_ref):
    z = (jnp.dot(x4_ref[...], c4_ref[...], preferred_element_type=jnp.float32)
         + g1_ref[...] + gs_ref[...] + b_ref[...])
    o_ref[...] = jnp.maximum(z, 0.0)


def _final(x4, g1, gsum, c4, b_mlp1, bm):
    mp = x4.shape[0]
    spec = pl.BlockSpec((bm, D), lambda i: (i, 0))
    return pl.pallas_call(
        _final_body,
        grid=(mp // bm,),
        in_specs=[spec, spec, spec,
                  pl.BlockSpec((D, D), lambda i: (0, 0)),
                  pl.BlockSpec((1, D), lambda i: (0, 0))],
        out_specs=spec,
        out_shape=jax.ShapeDtypeStruct((mp, D), jnp.float32),
    )(x4, g1, gsum, c4, b_mlp1)


_MESH = plsc.VectorSubcoreMesh(core_axis_name="c", subcore_axis_name="s",
                               num_cores=NC, num_subcores=NS)
_SC_PARAMS = pltpu.CompilerParams(needs_layout_passes=False)


def _seg1_kernel(mp):
    nb = mp // (NS * B)
    tpr = ND // NS

    @functools.partial(
        pl.kernel,
        out_type=[
            jax.ShapeDtypeStruct((ND, D), jnp.float32),
            jax.ShapeDtypeStruct((ND, D), jnp.float32),
            jax.ShapeDtypeStruct((ND, D), jnp.float32),
        ],
        mesh=_MESH,
        compiler_params=_SC_PARAMS,
        scratch_types=[
            pltpu.VMEM((B, D), jnp.float32),
            pltpu.VMEM((B,), jnp.int32),
            pltpu.VMEM_SHARED((ND, D), jnp.float32),
            pltpu.SemaphoreType.DMA,
        ],
    )
    def k(x_hbm, subg_hbm, node_hbm, dpos_hbm, ss_hbm, sn_hbm, dv_hbm,
          xbuf, ibuf, table, sem):
        cid = lax.axis_index("c")
        tid = lax.axis_index("s")
        _fill_zero(xbuf)
        _zero_rows(xbuf, table, tid * tpr, tpr)
        plsc.subcore_barrier()
        rpt = mp // NS

        @pl.when(cid == 0)
        def _():
            def body(b, _):
                base = tid * rpt + b * B
                pltpu.sync_copy(subg_hbm.at[pl.ds(base, B)], ibuf)
                pltpu.sync_copy(x_hbm.at[pl.ds(base, B), :], xbuf)
                pltpu.sync_copy(xbuf, table.at[ibuf], add=True)
                return 0
            lax.fori_loop(0, nb, body, 0)

        @pl.when(cid == 1)
        def _():
            def body(b, _):
                base = tid * rpt + b * B
                pltpu.sync_copy(node_hbm.at[pl.ds(base, B)], ibuf)
                pltpu.sync_copy(x_hbm.at[pl.ds(base, B), :], xbuf)
                pltpu.sync_copy(xbuf, table.at[ibuf], add=True)
                return 0
            lax.fori_loop(0, nb, body, 0)

        plsc.subcore_barrier()

        @pl.when(cid == 0)
        def _():
            pltpu.sync_copy(table.at[pl.ds(tid * tpr, tpr)],
                            ss_hbm.at[pl.ds(tid * tpr, tpr)])

            def dbody(b, _):
                base = tid * tpr + b * B
                pltpu.sync_copy(dpos_hbm.at[pl.ds(base, B)], ibuf)
                pltpu.async_copy(x_hbm.at[ibuf], xbuf, sem).wait()
                pltpu.sync_copy(xbuf, dv_hbm.at[pl.ds(base, B), :])
                return 0
            lax.fori_loop(0, tpr // B, dbody, 0)

        @pl.when(cid == 1)
        def _():
            pltpu.sync_copy(table.at[pl.ds(tid * tpr, tpr)],
                            sn_hbm.at[pl.ds(tid * tpr, tpr)])

    return k


def _x4_kernel(mp, n_slabs, nbounds):
    @functools.partial(
        pl.kernel,
        out_type=jax.ShapeDtypeStruct((mp, D), jnp.float32),
        mesh=_MESH,
        compiler_params=_SC_PARAMS,
        scratch_types=[
            pltpu.VMEM((B, D), jnp.float32),
            pltpu.VMEM((B,), jnp.int32),
            pltpu.VMEM((B,), jnp.int32),
            pltpu.VMEM((B,), jnp.int32),
            pltpu.VMEM((nbounds,), jnp.int32),
            pltpu.VMEM((B, D), jnp.float32),
            pltpu.VMEM_SHARED((SLABP, D), jnp.float32),
            pltpu.SemaphoreType.DMA,
        ],
    )
    def k(h_hbm, te0_hbm, te1_hbm, bounds_hbm, x4_hbm,
          rowbuf, sbuf, dbuf, lbuf, bbuf, zbuf, slab, sem):
        cid = lax.axis_index("c")
        tid = lax.axis_index("s")
        _fill_zero(zbuf)
        pltpu.sync_copy(bounds_hbm, bbuf)
        spt = SLABP // NS
        iota = _iota16()

        def do_slab(si):
            slab_base = si * SLAB
            _zero_rows(zbuf, slab, tid * spt, spt)
            plsc.subcore_barrier()

            e0 = _vmem_scalar(bbuf, si)
            e1 = _vmem_scalar(bbuf, si + 1)
            per = (e1 - e0 + NS - 1) // NS
            my0 = e0 + tid * per
            my1 = jnp.minimum(my0 + per, e1)
            my0a = (my0 // 8) * 8
            nbatch = jnp.maximum((my1 - my0a + B - 1) // B, 0)

            def batch(bi, _):
                eb = my0a + bi * B
                pltpu.sync_copy(te0_hbm.at[pl.ds(eb, B)], sbuf)
                pltpu.sync_copy(te1_hbm.at[pl.ds(eb, B)], dbuf)
                pltpu.async_copy(h_hbm.at[sbuf], rowbuf, sem).wait()
                for j in range(B // L):
                    dd = dbuf[pl.ds(j * L, L)]
                    e_ids = eb + j * L + iota
                    valid = (e_ids >= my0) & (e_ids < my1)
                    lbuf[pl.ds(j * L, L)] = jnp.where(valid, dd - slab_base,
                                                      jnp.int32(SLAB))
                pltpu.sync_copy(rowbuf, slab.at[lbuf], add=True)
                return 0

            lax.fori_loop(0, nbatch, batch, 0)
            plsc.subcore_barrier()
            fpt = SLAB // NS
            pltpu.sync_copy(slab.at[pl.ds(tid * fpt, fpt)],
                            x4_hbm.at[pl.ds(slab_base + tid * fpt, fpt)])
            plsc.subcore_barrier()

        nmine = (n_slabs + 1) // 2

        def mine(i, _):
            si = i * 2 + cid

            @pl.when(si < n_slabs)
            def _():
                do_slab(si)
            return 0

        lax.fori_loop(0, nmine, mine, 0)

    return k


def _seg7_kernel(mp):
    half = mp // 2
    nb = half // (NS * B)
    tpr = ND // NS

    @functools.partial(
        pl.kernel,
        out_type=[
            jax.ShapeDtypeStruct((2, ND, D), jnp.float32),
            jax.ShapeDtypeStruct((ND, D), jnp.float32),
        ],
        mesh=_MESH,
        compiler_params=_SC_PARAMS,
        scratch_types=[
            pltpu.VMEM((B, D), jnp.float32),
            pltpu.VMEM((B,), jnp.int32),
            pltpu.VMEM_SHARED((ND, D), jnp.float32),
            pltpu.SemaphoreType.DMA,
        ],
    )
    def k(x4_hbm, node_hbm, dpos_hbm, s7_hbm, x4d_hbm, xbuf, ibuf, table, sem):
        cid = lax.axis_index("c")
        tid = lax.axis_index("s")
        _fill_zero(xbuf)
        _zero_rows(xbuf, table, tid * tpr, tpr)
        plsc.subcore_barrier()
        rpt = half // NS

        def body(b, _):
            base = cid * half + tid * rpt + b * B
            pltpu.sync_copy(node_hbm.at[pl.ds(base, B)], ibuf)
            pltpu.sync_copy(x4_hbm.at[pl.ds(base, B), :], xbuf)
            pltpu.sync_copy(xbuf, table.at[ibuf], add=True)
            return 0

        lax.fori_loop(0, nb, body, 0)
        plsc.subcore_barrier()
        pltpu.sync_copy(table.at[pl.ds(tid * tpr, tpr)],
                        s7_hbm.at[cid, pl.ds(tid * tpr, tpr), :])

        @pl.when(cid == 0)
        def _():
            def dbody(b, _):
                base = tid * tpr + b * B
                pltpu.sync_copy(dpos_hbm.at[pl.ds(base, B)], ibuf)
                pltpu.async_copy(x4_hbm.at[ibuf], xbuf, sem).wait()
                pltpu.sync_copy(xbuf, x4d_hbm.at[pl.ds(base, B), :])
                return 0
            lax.fori_loop(0, tpr // B, dbody, 0)

    return k


def _gsum_kernel(mp):
    nw = NC * NS
    rpw = mp // nw
    nb = rpw // B

    @functools.partial(
        pl.kernel,
        out_type=jax.ShapeDtypeStruct((mp, D), jnp.float32),
        mesh=_MESH,
        compiler_params=_SC_PARAMS,
        scratch_types=[
            pltpu.VMEM((B, D), jnp.float32),
            pltpu.VMEM((B,), jnp.int32),
            pltpu.SemaphoreType.DMA,
        ],
    )
    def k(ts_hbm, tn_hbm, p7t_hbm, dl_hbm, subg_hbm, node_hbm, dsel_hbm,
          g_hbm, buf, ibuf, sem):
        cid = lax.axis_index("c")
        tid = lax.axis_index("s")
        wid = tid * NC + cid

        def body(b, _):
            base = wid * rpw + b * B
            pltpu.sync_copy(subg_hbm.at[pl.ds(base, B)], ibuf)
            pltpu.async_copy(ts_hbm.at[ibuf], buf, sem).wait()
            pltpu.sync_copy(node_hbm.at[pl.ds(base, B)], ibuf)
            pltpu.async_copy(tn_hbm.at[ibuf], buf, sem, add=True).wait()
            pltpu.async_copy(p7t_hbm.at[ibuf], buf, sem, add=True).wait()
            pltpu.sync_copy(dsel_hbm.at[pl.ds(base, B)], ibuf)
            pltpu.async_copy(dl_hbm.at[ibuf], buf, sem, add=True).wait()
            pltpu.sync_copy(buf, g_hbm.at[pl.ds(base, B), :])
            return 0

        lax.fori_loop(0, nb, body, 0)

    return k


def kernel(X, W_lin0, b_lin0, W_het, W_mlp1, b_mlp1, subg_idx, node_idx, tuple_edge):
    m = X.shape[0]
    k_edges = tuple_edge.shape[1]

    chunk = 2 * SLAB
    mp = ((m + chunk - 1) // chunk) * chunk

    subg = subg_idx.astype(jnp.int32)
    node = node_idx.astype(jnp.int32)
    te0 = tuple_edge[0].astype(jnp.int32)
    te1 = tuple_edge[1].astype(jnp.int32)

    xp = jnp.pad(X, ((0, mp - m), (0, 0)))
    subg_p = jnp.pad(subg, (0, mp - m), constant_values=N)
    node_p = jnp.pad(node, (0, mp - m), constant_values=N)
    dsel_p = jnp.pad(jnp.where(subg == node, subg, N), (0, mp - m), constant_values=N)
    kchunk = NC * NS * B
    kp = ((k_edges + 2 * B + kchunk - 1) // kchunk) * kchunk
    te0_p = jnp.pad(te0, (0, kp - k_edges), constant_values=0)
    te1_p = jnp.pad(te1, (0, kp - k_edges), constant_values=0)

    codes = subg * N + node
    dpos = jnp.searchsorted(codes, jnp.arange(N, dtype=jnp.int32) * (N + 1)).astype(jnp.int32)
    dpos_p = jnp.pad(dpos, (0, ND - N), constant_values=0)

    n_slabs = (mp + SLAB - 1) // SLAB
    nbounds = ((n_slabs + 1 + L - 1) // L) * L
    slab_bounds = jnp.searchsorted(te1, jnp.arange(n_slabs + 1, dtype=jnp.int32) * SLAB).astype(jnp.int32)
    slab_bounds = jnp.pad(slab_bounds, (0, nbounds - n_slabs - 1), constant_values=k_edges)

    bnd = jnp.arange(N + 1, dtype=jnp.int32)
    cnt_s1 = jnp.diff(jnp.searchsorted(subg, bnd)).astype(jnp.float32)
    cnt_n1 = jnp.diff(jnp.searchsorted(jnp.sort(node), bnd)).astype(jnp.float32)
    rcp_s = jnp.broadcast_to(jnp.pad(1.0 / jnp.maximum(cnt_s1, 1.0), (0, ND - N))[:, None], (ND, D))
    rcp_n = jnp.broadcast_to(jnp.pad(1.0 / jnp.maximum(cnt_n1, 1.0), (0, ND - N))[:, None], (ND, D))

    call = _fold_weights(W_het.reshape(2 * 7 * D, D), W_mlp1).reshape(2, 7, D, D)
    c1 = call[0, 0]
    c4 = call[0, 3]
    b0 = b_lin0.reshape(1, D)
    bm1 = b_mlp1.reshape(1, D)

    h, g1 = _pass1(xp, W_lin0, b0, c1, 512)

    s_s, s_n, dvec = _seg1_kernel(mp)(xp, subg_p, node_p, dpos_p)

    x4 = _x4_kernel(mp, n_slabs, nbounds)(h, te0_p, te1_p, slab_bounds)

    s7p, x4d = _seg7_kernel(mp)(x4, node_p, dpos_p)

    ts, tn, p7t, delta = _tables(s_s, s_n, s7p, dvec, x4d, rcp_s, rcp_n, call)

    gsum = _gsum_kernel(mp)(ts, tn, p7t, delta, subg_p, node_p, dsel_p)

    out = _final(x4, g1, gsum, c4, bm1, 512)
    return out[:m]


# speedup vs baseline: 1.0049x; 1.0049x over previous
import functools
import jax
import jax.numpy as jnp
from jax import lax
from jax.experimental import pallas as pl
from jax.experimental.pallas import tpu as pltpu
from jax.experimental.pallas import tpu_sc as plsc

N = 10000
D = 128
NC = 2
NS = 16
L = 16
B = 64
ND = 10240
SLAB = 8192
SLABP = SLAB + 128


def _iota16():
    return lax.iota(jnp.int32, 16)


def _vmem_scalar(ref, j):
    base = (j // L) * L
    v = ref[pl.ds(base, L)]
    m = _iota16() == (j - base)
    return jnp.max(jnp.where(m, v, jnp.int32(0)))


def _zero_rows(zbuf, dst, row0, nrows):
    h = zbuf.shape[0]
    nfull = nrows // h
    for k in range(nfull):
        pltpu.sync_copy(zbuf, dst.at[pl.ds(row0 + k * h, h)])
    rem = nrows - nfull * h
    if rem > 0:
        pltpu.sync_copy(zbuf.at[pl.ds(0, rem)], dst.at[pl.ds(row0 + nfull * h, rem)])


def _fill_zero(buf):
    rows, cols = buf.shape
    z = jnp.zeros((L,), jnp.float32)

    def row_body(i, _):
        for c in range(cols // L):
            buf[i, pl.ds(c * L, L)] = z
        return 0

    lax.fori_loop(0, rows, row_body, 0)


def _mm_body(a_ref, b_ref, o_ref):
    o_ref[...] = jnp.dot(a_ref[...], b_ref[...], preferred_element_type=jnp.float32)


def _fold_weights(whet_flat, w_mlp1):
    return pl.pallas_call(
        _mm_body,
        out_shape=jax.ShapeDtypeStruct((whet_flat.shape[0], D), jnp.float32),
    )(whet_flat, w_mlp1)


def _pass1_body(x_ref, w_ref, b_ref, c1_ref, h_ref, g1_ref):
    x = x_ref[...]
    h_ref[...] = jnp.maximum(jnp.dot(x, w_ref[...], preferred_element_type=jnp.float32)
                             + b_ref[...], 0.0)
    g1_ref[...] = jnp.dot(x, c1_ref[...], preferred_element_type=jnp.float32)


def _pass1(xp, w_lin0, b_lin0, c1, bm):
    mp = xp.shape[0]
    return pl.pallas_call(
        _pass1_body,
        grid=(mp // bm,),
        in_specs=[
            pl.BlockSpec((bm, D), lambda i: (i, 0)),
            pl.BlockSpec((D, D), lambda i: (0, 0)),
            pl.BlockSpec((1, D), lambda i: (0, 0)),
            pl.BlockSpec((D, D), lambda i: (0, 0)),
        ],
        out_specs=[
            pl.BlockSpec((bm, D), lambda i: (i, 0)),
            pl.BlockSpec((bm, D), lambda i: (i, 0)),
        ],
        out_shape=[
            jax.ShapeDtypeStruct((mp, D), jnp.float32),
            jax.ShapeDtypeStruct((mp, D), jnp.float32),
        ],
    )(xp, w_lin0, b_lin0, c1)


def _tables_body(ss_ref, sn_ref, s7_ref, dv_ref, x4d_ref, rs_ref, rn_ref,
                 c_ref, ts_ref, tn_ref, dl_ref, *, bn):
    pid = pl.program_id(0)
    rid = pid * bn + lax.broadcasted_iota(jnp.int32, (bn, 1), 0)
    m = (rid < N).astype(jnp.float32)
    rcp_s = rs_ref[...] * m
    rcp_n = rn_ref[...] * m
    dv = dv_ref[...] * m
    x4d = x4d_ref[...] * m
    p5 = sn_ref[...] * rcp_n
    p6 = ss_ref[...] * rcp_s
    p7 = (s7_ref[0] + s7_ref[1]) * rcp_n
    c = c_ref[...]

    def mm(a, w):
        return jnp.dot(a, w, preferred_element_type=jnp.float32)

    ts = mm(dv, c[0, 1]) + mm(p6, c[0, 5])
    tn = mm(dv, c[0, 2]) + mm(p5, c[0, 4])
    p7t = mm(p7, c[0, 6])
    zd = (mm(dv, c[1, 0] + c[1, 1] + c[1, 2]) + mm(x4d, c[1, 3])
          + mm(p5, c[1, 4]) + mm(p6, c[1, 5]) + mm(p7, c[1, 6]))
    base_diag = mm(x4d, c[0, 3]) + mm(dv, c[0, 0]) + ts + tn + p7t
    ts_ref[...] = ts * m
    tn_ref[...] = (tn + p7t) * m
    dl_ref[...] = (zd - base_diag) * m


def _tables(s_s, s_n, s7p, dvec, x4d, rcp_s, rcp_n, call):
    bn = 1024
    spec = pl.BlockSpec((bn, D), lambda i: (i, 0))
    out = jax.ShapeDtypeStruct((ND, D), jnp.float32)
    return pl.pallas_call(
        functools.partial(_tables_body, bn=bn),
        grid=(ND // bn,),
        in_specs=[
            spec, spec,
            pl.BlockSpec((2, bn, D), lambda i: (0, i, 0)),
            spec, spec, spec, spec,
            pl.BlockSpec((2, 7, D, D), lambda i: (0, 0, 0, 0)),
        ],
        out_specs=[spec, spec, spec],
        out_shape=[out, out, out],
    )(s_s, s_n, s7p, dvec, x4d, rcp_s, rcp_n, call)


def _final_body(x4_ref, g1_ref, gs_ref, c4_ref, b_ref, o_ref):
    z = (jnp.dot(x4_ref[...], c4_ref[...], preferred_element_type=jnp.float32)
         + g1_ref[...] + gs_ref[...] + b_ref[...])
    o_ref[...] = jnp.maximum(z, 0.0)


def _final(x4, g1, gsum, c4, b_mlp1, bm):
    mp = x4.shape[0]
    spec = pl.BlockSpec((bm, D), lambda i: (i, 0))
    return pl.pallas_call(
        _final_body,
        grid=(mp // bm,),
        in_specs=[spec, spec, spec,
                  pl.BlockSpec((D, D), lambda i: (0, 0)),
                  pl.BlockSpec((1, D), lambda i: (0, 0))],
        out_specs=spec,
        out_shape=jax.ShapeDtypeStruct((mp, D), jnp.float32),
    )(x4, g1, gsum, c4, b_mlp1)


_MESH = plsc.VectorSubcoreMesh(core_axis_name="c", subcore_axis_name="s",
                               num_cores=NC, num_subcores=NS)
_SC_PARAMS = pltpu.CompilerParams(needs_layout_passes=False)


def _seg1_kernel(mp):
    nb = mp // (NS * B)
    tpr = ND // NS

    @functools.partial(
        pl.kernel,
        out_type=[
            jax.ShapeDtypeStruct((ND, D), jnp.float32),
            jax.ShapeDtypeStruct((ND, D), jnp.float32),
            jax.ShapeDtypeStruct((ND, D), jnp.float32),
        ],
        mesh=_MESH,
        compiler_params=_SC_PARAMS,
        scratch_types=[
            pltpu.VMEM((B, D), jnp.float32),
            pltpu.VMEM((B,), jnp.int32),
            pltpu.VMEM_SHARED((ND, D), jnp.float32),
            pltpu.SemaphoreType.DMA,
        ],
    )
    def k(x_hbm, subg_hbm, node_hbm, dpos_hbm, ss_hbm, sn_hbm, dv_hbm,
          xbuf, ibuf, table, sem):
        cid = lax.axis_index("c")
        tid = lax.axis_index("s")
        _fill_zero(xbuf)
        _zero_rows(xbuf, table, tid * tpr, tpr)
        plsc.subcore_barrier()
        rpt = mp // NS

        @pl.when(cid == 0)
        def _():
            def body(b, _):
                base = tid * rpt + b * B
                pltpu.sync_copy(subg_hbm.at[pl.ds(base, B)], ibuf)
                pltpu.sync_copy(x_hbm.at[pl.ds(base, B), :], xbuf)
                pltpu.sync_copy(xbuf, table.at[ibuf], add=True)
                return 0
            lax.fori_loop(0, nb, body, 0)

        @pl.when(cid == 1)
        def _():
            def body(b, _):
                base = tid * rpt + b * B
                pltpu.sync_copy(node_hbm.at[pl.ds(base, B)], ibuf)
                pltpu.sync_copy(x_hbm.at[pl.ds(base, B), :], xbuf)
                pltpu.sync_copy(xbuf, table.at[ibuf], add=True)
                return 0
            lax.fori_loop(0, nb, body, 0)

        plsc.subcore_barrier()

        @pl.when(cid == 0)
        def _():
            pltpu.sync_copy(table.at[pl.ds(tid * tpr, tpr)],
                            ss_hbm.at[pl.ds(tid * tpr, tpr)])

            def dbody(b, _):
                base = tid * tpr + b * B
                pltpu.sync_copy(dpos_hbm.at[pl.ds(base, B)], ibuf)
                pltpu.async_copy(x_hbm.at[ibuf], xbuf, sem).wait()
                pltpu.sync_copy(xbuf, dv_hbm.at[pl.ds(base, B), :])
                return 0
            lax.fori_loop(0, tpr // B, dbody, 0)

        @pl.when(cid == 1)
        def _():
            pltpu.sync_copy(table.at[pl.ds(tid * tpr, tpr)],
                            sn_hbm.at[pl.ds(tid * tpr, tpr)])

    return k


def _x4_kernel(mp, n_slabs, nbounds):
    @functools.partial(
        pl.kernel,
        out_type=jax.ShapeDtypeStruct((mp, D), jnp.float32),
        mesh=_MESH,
        compiler_params=_SC_PARAMS,
        scratch_types=[
            pltpu.VMEM((B, D), jnp.float32),
            pltpu.VMEM((B,), jnp.int32),
            pltpu.VMEM((B,), jnp.int32),
            pltpu.VMEM((B,), jnp.int32),
            pltpu.VMEM((nbounds,), jnp.int32),
            pltpu.VMEM((B, D), jnp.float32),
            pltpu.VMEM_SHARED((SLABP, D), jnp.float32),
            pltpu.SemaphoreType.DMA,
        ],
    )
    def k(h_hbm, te0_hbm, te1_hbm, bounds_hbm, x4_hbm,
          rowbuf, sbuf, dbuf, lbuf, bbuf, zbuf, slab, sem):
        cid = lax.axis_index("c")
        tid = lax.axis_index("s")
        _fill_zero(zbuf)
        pltpu.sync_copy(bounds_hbm, bbuf)
        spt = SLABP // NS
        iota = _iota16()

        def do_slab(si):
            slab_base = si * SLAB
            _zero_rows(zbuf, slab, tid * spt, spt)
            plsc.subcore_barrier()

            e0 = _vmem_scalar(bbuf, si)
            e1 = _vmem_scalar(bbuf, si + 1)
            per = (e1 - e0 + NS - 1) // NS
            my0 = e0 + tid * per
            my1 = jnp.minimum(my0 + per, e1)
            my0a = (my0 // 8) * 8
            nbatch = jnp.maximum((my1 - my0a + B - 1) // B, 0)

            def batch(bi, _):
                eb = my0a + bi * B
                pltpu.sync_copy(te0_hbm.at[pl.ds(eb, B)], sbuf)
                pltpu.sync_copy(te1_hbm.at[pl.ds(eb, B)], dbuf)
                pltpu.async_copy(h_hbm.at[sbuf], rowbuf, sem).wait()
                for j in range(B // L):
                    dd = dbuf[pl.ds(j * L, L)]
                    e_ids = eb + j * L + iota
                    valid = (e_ids >= my0) & (e_ids < my1)
                    lbuf[pl.ds(j * L, L)] = jnp.where(valid, dd - slab_base,
                                                      jnp.int32(SLAB))
                pltpu.sync_copy(rowbuf, slab.at[lbuf], add=True)
                return 0

            lax.fori_loop(0, nbatch, batch, 0)
            plsc.subcore_barrier()
            fpt = SLAB // NS
            pltpu.sync_copy(slab.at[pl.ds(tid * fpt, fpt)],
                            x4_hbm.at[pl.ds(slab_base + tid * fpt, fpt)])
            plsc.subcore_barrier()

        nmine = (n_slabs + 1) // 2

        def mine(i, _):
            si = i * 2 + cid

            @pl.when(si < n_slabs)
            def _():
                do_slab(si)
            return 0

        lax.fori_loop(0, nmine, mine, 0)

    return k


def _seg7_kernel(mp):
    half = mp // 2
    nb = half // (NS * B)
    tpr = ND // NS

    @functools.partial(
        pl.kernel,
        out_type=[
            jax.ShapeDtypeStruct((2, ND, D), jnp.float32),
            jax.ShapeDtypeStruct((ND, D), jnp.float32),
        ],
        mesh=_MESH,
        compiler_params=_SC_PARAMS,
        scratch_types=[
            pltpu.VMEM((B, D), jnp.float32),
            pltpu.VMEM((B,), jnp.int32),
            pltpu.VMEM_SHARED((ND, D), jnp.float32),
            pltpu.SemaphoreType.DMA,
        ],
    )
    def k(x4_hbm, node_hbm, dpos_hbm, s7_hbm, x4d_hbm, xbuf, ibuf, table, sem):
        cid = lax.axis_index("c")
        tid = lax.axis_index("s")
        _fill_zero(xbuf)
        _zero_rows(xbuf, table, tid * tpr, tpr)
        plsc.subcore_barrier()
        rpt = half // NS

        def body(b, _):
            base = cid * half + tid * rpt + b * B
            pltpu.sync_copy(node_hbm.at[pl.ds(base, B)], ibuf)
            pltpu.sync_copy(x4_hbm.at[pl.ds(base, B), :], xbuf)
            pltpu.sync_copy(xbuf, table.at[ibuf], add=True)
            return 0

        lax.fori_loop(0, nb, body, 0)
        plsc.subcore_barrier()
        pltpu.sync_copy(table.at[pl.ds(tid * tpr, tpr)],
                        s7_hbm.at[cid, pl.ds(tid * tpr, tpr), :])

        @pl.when(cid == 0)
        def _():
            def dbody(b, _):
                base = tid * tpr + b * B
                pltpu.sync_copy(dpos_hbm.at[pl.ds(base, B)], ibuf)
                pltpu.async_copy(x4_hbm.at[ibuf], xbuf, sem).wait()
                pltpu.sync_copy(xbuf, x4d_hbm.at[pl.ds(base, B), :])
                return 0
            lax.fori_loop(0, tpr // B, dbody, 0)

    return k


BG = 256


def _gsum_kernel(mp):
    nw = NC * NS
    rpw = mp // nw
    nb = rpw // BG

    @functools.partial(
        pl.kernel,
        out_type=jax.ShapeDtypeStruct((mp, D), jnp.float32),
        mesh=_MESH,
        compiler_params=_SC_PARAMS,
        scratch_types=[
            pltpu.VMEM((BG, D), jnp.float32),
            pltpu.VMEM((3 * BG,), jnp.int32),
            pltpu.SemaphoreType.DMA,
        ],
    )
    def k(ts_hbm, tn_hbm, dl_hbm, pack_hbm, g_hbm, buf, ibuf, sem):
        cid = lax.axis_index("c")
        tid = lax.axis_index("s")
        wid = tid * NC + cid

        def body(b, _):
            g = wid * nb + b
            base = wid * rpw + b * BG
            pltpu.sync_copy(pack_hbm.at[pl.ds(g * 3 * BG, 3 * BG)], ibuf)
            pltpu.async_copy(ts_hbm.at[ibuf.at[pl.ds(0, BG)]], buf, sem).wait()
            pltpu.async_copy(tn_hbm.at[ibuf.at[pl.ds(BG, BG)]], buf, sem, add=True).wait()
            pltpu.async_copy(dl_hbm.at[ibuf.at[pl.ds(2 * BG, BG)]], buf, sem, add=True).wait()
            pltpu.sync_copy(buf, g_hbm.at[pl.ds(base, BG), :])
            return 0

        lax.fori_loop(0, nb, body, 0)

    return k


def kernel(X, W_lin0, b_lin0, W_het, W_mlp1, b_mlp1, subg_idx, node_idx, tuple_edge):
    m = X.shape[0]
    k_edges = tuple_edge.shape[1]

    chunk = 2 * SLAB
    mp = ((m + chunk - 1) // chunk) * chunk

    subg = subg_idx.astype(jnp.int32)
    node = node_idx.astype(jnp.int32)
    te0 = tuple_edge[0].astype(jnp.int32)
    te1 = tuple_edge[1].astype(jnp.int32)

    xp = jnp.pad(X, ((0, mp - m), (0, 0)))
    subg_p = jnp.pad(subg, (0, mp - m), constant_values=N)
    node_p = jnp.pad(node, (0, mp - m), constant_values=N)
    dsel_p = jnp.pad(jnp.where(subg == node, subg, N), (0, mp - m), constant_values=N)
    kchunk = NC * NS * B
    kp = ((k_edges + 2 * B + kchunk - 1) // kchunk) * kchunk
    te0_p = jnp.pad(te0, (0, kp - k_edges), constant_values=0)
    te1_p = jnp.pad(te1, (0, kp - k_edges), constant_values=0)

    codes = subg * N + node
    dpos = jnp.searchsorted(codes, jnp.arange(N, dtype=jnp.int32) * (N + 1)).astype(jnp.int32)
    dpos_p = jnp.pad(dpos, (0, ND - N), constant_values=0)

    n_slabs = (mp + SLAB - 1) // SLAB
    nbounds = ((n_slabs + 1 + L - 1) // L) * L
    slab_bounds = jnp.searchsorted(te1, jnp.arange(n_slabs + 1, dtype=jnp.int32) * SLAB).astype(jnp.int32)
    slab_bounds = jnp.pad(slab_bounds, (0, nbounds - n_slabs - 1), constant_values=k_edges)

    bnd = jnp.arange(N + 1, dtype=jnp.int32)
    cnt_s1 = jnp.diff(jnp.searchsorted(subg, bnd)).astype(jnp.float32)
    cnt_n1 = jnp.diff(jnp.searchsorted(jnp.sort(node), bnd)).astype(jnp.float32)
    rcp_s = jnp.broadcast_to(jnp.pad(1.0 / jnp.maximum(cnt_s1, 1.0), (0, ND - N))[:, None], (ND, D))
    rcp_n = jnp.broadcast_to(jnp.pad(1.0 / jnp.maximum(cnt_n1, 1.0), (0, ND - N))[:, None], (ND, D))

    call = _fold_weights(W_het.reshape(2 * 7 * D, D), W_mlp1).reshape(2, 7, D, D)
    c1 = call[0, 0]
    c4 = call[0, 3]
    b0 = b_lin0.reshape(1, D)
    bm1 = b_mlp1.reshape(1, D)

    h, g1 = _pass1(xp, W_lin0, b0, c1, 512)

    s_s, s_n, dvec = _seg1_kernel(mp)(xp, subg_p, node_p, dpos_p)

    x4 = _x4_kernel(mp, n_slabs, nbounds)(h, te0_p, te1_p, slab_bounds)

    s7p, x4d = _seg7_kernel(mp)(x4, node_p, dpos_p)

    ts, tnp, delta = _tables(s_s, s_n, s7p, dvec, x4d, rcp_s, rcp_n, call)

    idx_pack = jnp.stack([subg_p.reshape(-1, BG), node_p.reshape(-1, BG),
                          dsel_p.reshape(-1, BG)], axis=1).reshape(-1)
    gsum = _gsum_kernel(mp)(ts, tnp, delta, idx_pack)

    out = _final(x4, g1, gsum, c4, bm1, 512)
    return out[:m]


# speedup vs baseline: 2.3640x; 2.3525x over previous
import functools
import jax
import jax.numpy as jnp
from jax import lax
from jax.experimental import pallas as pl
from jax.experimental.pallas import tpu as pltpu
from jax.experimental.pallas import tpu_sc as plsc

N = 10000
D = 128
NC = 2
NS = 16
L = 16
B = 64
ND = 10240
SLAB = 8192
SLABP = SLAB + 128


def _iota16():
    return lax.iota(jnp.int32, 16)


def _vmem_scalar(ref, j):
    base = (j // L) * L
    v = ref[pl.ds(base, L)]
    m = _iota16() == (j - base)
    return jnp.max(jnp.where(m, v, jnp.int32(0)))


def _zero_rows(zbuf, dst, row0, nrows):
    h = zbuf.shape[0]
    nfull = nrows // h
    for k in range(nfull):
        pltpu.sync_copy(zbuf, dst.at[pl.ds(row0 + k * h, h)])
    rem = nrows - nfull * h
    if rem > 0:
        pltpu.sync_copy(zbuf.at[pl.ds(0, rem)], dst.at[pl.ds(row0 + nfull * h, rem)])


def _fill_zero(buf):
    rows, cols = buf.shape
    z = jnp.zeros((L,), jnp.float32)

    def row_body(i, _):
        for c in range(cols // L):
            buf[i, pl.ds(c * L, L)] = z
        return 0

    lax.fori_loop(0, rows, row_body, 0)


def _mm_body(a_ref, b_ref, o_ref):
    o_ref[...] = jnp.dot(a_ref[...], b_ref[...], preferred_element_type=jnp.float32)


def _fold_weights(whet_flat, w_mlp1):
    return pl.pallas_call(
        _mm_body,
        out_shape=jax.ShapeDtypeStruct((whet_flat.shape[0], D), jnp.float32),
    )(whet_flat, w_mlp1)


def _pass1_body(x_ref, w_ref, b_ref, c1_ref, h_ref, g1_ref):
    x = x_ref[...]
    h_ref[...] = jnp.maximum(jnp.dot(x, w_ref[...], preferred_element_type=jnp.float32)
                             + b_ref[...], 0.0)
    g1_ref[...] = jnp.dot(x, c1_ref[...], preferred_element_type=jnp.float32)


def _pass1(xp, w_lin0, b_lin0, c1, bm):
    mp = xp.shape[0]
    return pl.pallas_call(
        _pass1_body,
        grid=(mp // bm,),
        in_specs=[
            pl.BlockSpec((bm, D), lambda i: (i, 0)),
            pl.BlockSpec((D, D), lambda i: (0, 0)),
            pl.BlockSpec((1, D), lambda i: (0, 0)),
            pl.BlockSpec((D, D), lambda i: (0, 0)),
        ],
        out_specs=[
            pl.BlockSpec((bm, D), lambda i: (i, 0)),
            pl.BlockSpec((bm, D), lambda i: (i, 0)),
        ],
        out_shape=[
            jax.ShapeDtypeStruct((mp, D), jnp.float32),
            jax.ShapeDtypeStruct((mp, D), jnp.float32),
        ],
    )(xp, w_lin0, b_lin0, c1)


def _tables_body(ss_ref, sn_ref, s7_ref, dv_ref, x4d_ref, rs_ref, rn_ref,
                 c_ref, ts_ref, tn_ref, td_ref, *, bn):
    pid = pl.program_id(0)
    rid = pid * bn + lax.broadcasted_iota(jnp.int32, (bn, 1), 0)
    m = (rid < N).astype(jnp.float32)
    rcp_s = rs_ref[...] * m
    rcp_n = rn_ref[...] * m
    dv = dv_ref[...] * m
    x4d = x4d_ref[...] * m
    p5 = sn_ref[...] * rcp_n
    p6 = ss_ref[...] * rcp_s
    p7 = (s7_ref[0] + s7_ref[1]) * rcp_n
    c = c_ref[...]

    def mm(a, w):
        return jnp.dot(a, w, preferred_element_type=jnp.float32)

    ts = mm(dv, c[0, 1]) + mm(p6, c[0, 5])
    tn = mm(dv, c[0, 2]) + mm(p5, c[0, 4])
    p7t = mm(p7, c[0, 6])
    zd = (mm(dv, c[1, 0] + c[1, 1] + c[1, 2]) + mm(x4d, c[1, 3])
          + mm(p5, c[1, 4]) + mm(p6, c[1, 5]) + mm(p7, c[1, 6]))
    base_diag = mm(x4d, c[0, 3]) + mm(dv, c[0, 0]) + ts + tn + p7t
    delta = zd - base_diag
    ts_ref[...] = ts * m
    tn_ref[...] = (tn + p7t) * m
    td_ref[...] = (ts + delta) * m


def _tables(s_s, s_n, s7p, dvec, x4d, rcp_s, rcp_n, call):
    bn = 1024
    spec = pl.BlockSpec((bn, D), lambda i: (i, 0))
    out = jax.ShapeDtypeStruct((ND, D), jnp.float32)
    return pl.pallas_call(
        functools.partial(_tables_body, bn=bn),
        grid=(ND // bn,),
        in_specs=[
            spec, spec,
            pl.BlockSpec((2, bn, D), lambda i: (0, i, 0)),
            spec, spec, spec, spec,
            pl.BlockSpec((2, 7, D, D), lambda i: (0, 0, 0, 0)),
        ],
        out_specs=[spec, spec, spec],
        out_shape=[out, out, out],
    )(s_s, s_n, s7p, dvec, x4d, rcp_s, rcp_n, call)


def _final_body(x4_ref, g1_ref, ga_ref, gb_ref, c4_ref, b_ref, o_ref):
    z = (jnp.dot(x4_ref[...], c4_ref[...], preferred_element_type=jnp.float32)
         + g1_ref[...] + ga_ref[...] + gb_ref[...] + b_ref[...])
    o_ref[...] = jnp.maximum(z, 0.0)


def _final(x4, g1, ga, gb, c4, b_mlp1, bm):
    mp = x4.shape[0]
    spec = pl.BlockSpec((bm, D), lambda i: (i, 0))
    return pl.pallas_call(
        _final_body,
        grid=(mp // bm,),
        in_specs=[spec, spec, spec, spec,
                  pl.BlockSpec((D, D), lambda i: (0, 0)),
                  pl.BlockSpec((1, D), lambda i: (0, 0))],
        out_specs=spec,
        out_shape=jax.ShapeDtypeStruct((mp, D), jnp.float32),
    )(x4, g1, ga, gb, c4, b_mlp1)


_MESH = plsc.VectorSubcoreMesh(core_axis_name="c", subcore_axis_name="s",
                               num_cores=NC, num_subcores=NS)
_SC_PARAMS = pltpu.CompilerParams(needs_layout_passes=False)


def _seg1_kernel(mp):
    nb = mp // (NS * B)
    tpr = ND // NS

    @functools.partial(
        pl.kernel,
        out_type=[
            jax.ShapeDtypeStruct((ND, D), jnp.float32),
            jax.ShapeDtypeStruct((ND, D), jnp.float32),
            jax.ShapeDtypeStruct((ND, D), jnp.float32),
        ],
        mesh=_MESH,
        compiler_params=_SC_PARAMS,
        scratch_types=[
            pltpu.VMEM((B, D), jnp.float32),
            pltpu.VMEM((B,), jnp.int32),
            pltpu.VMEM_SHARED((ND, D), jnp.float32),
            pltpu.SemaphoreType.DMA,
        ],
    )
    def k(x_hbm, subg_hbm, node_hbm, dpos_hbm, ss_hbm, sn_hbm, dv_hbm,
          xbuf, ibuf, table, sem):
        cid = lax.axis_index("c")
        tid = lax.axis_index("s")
        _fill_zero(xbuf)
        _zero_rows(xbuf, table, tid * tpr, tpr)
        plsc.subcore_barrier()
        rpt = mp // NS

        @pl.when(cid == 0)
        def _():
            def body(b, _):
                base = tid * rpt + b * B
                pltpu.sync_copy(subg_hbm.at[pl.ds(base, B)], ibuf)
                pltpu.sync_copy(x_hbm.at[pl.ds(base, B), :], xbuf)
                pltpu.sync_copy(xbuf, table.at[ibuf], add=True)
                return 0
            lax.fori_loop(0, nb, body, 0)

        @pl.when(cid == 1)
        def _():
            def body(b, _):
                base = tid * rpt + b * B
                pltpu.sync_copy(node_hbm.at[pl.ds(base, B)], ibuf)
                pltpu.sync_copy(x_hbm.at[pl.ds(base, B), :], xbuf)
                pltpu.sync_copy(xbuf, table.at[ibuf], add=True)
                return 0
            lax.fori_loop(0, nb, body, 0)

        plsc.subcore_barrier()

        @pl.when(cid == 0)
        def _():
            pltpu.sync_copy(table.at[pl.ds(tid * tpr, tpr)],
                            ss_hbm.at[pl.ds(tid * tpr, tpr)])

            def dbody(b, _):
                base = tid * tpr + b * B
                pltpu.sync_copy(dpos_hbm.at[pl.ds(base, B)], ibuf)
                pltpu.async_copy(x_hbm.at[ibuf], xbuf, sem).wait()
                pltpu.sync_copy(xbuf, dv_hbm.at[pl.ds(base, B), :])
                return 0
            lax.fori_loop(0, tpr // B, dbody, 0)

        @pl.when(cid == 1)
        def _():
            pltpu.sync_copy(table.at[pl.ds(tid * tpr, tpr)],
                            sn_hbm.at[pl.ds(tid * tpr, tpr)])

    return k


def _x4_kernel(mp, n_slabs, nbounds):
    @functools.partial(
        pl.kernel,
        out_type=jax.ShapeDtypeStruct((mp, D), jnp.float32),
        mesh=_MESH,
        compiler_params=_SC_PARAMS,
        scratch_types=[
            pltpu.VMEM((B, D), jnp.float32),
            pltpu.VMEM((B,), jnp.int32),
            pltpu.VMEM((B,), jnp.int32),
            pltpu.VMEM((B,), jnp.int32),
            pltpu.VMEM((nbounds,), jnp.int32),
            pltpu.VMEM((B, D), jnp.float32),
            pltpu.VMEM_SHARED((SLABP, D), jnp.float32),
            pltpu.SemaphoreType.DMA,
        ],
    )
    def k(h_hbm, te0_hbm, te1_hbm, bounds_hbm, x4_hbm,
          rowbuf, sbuf, dbuf, lbuf, bbuf, zbuf, slab, sem):
        cid = lax.axis_index("c")
        tid = lax.axis_index("s")
        _fill_zero(zbuf)
        pltpu.sync_copy(bounds_hbm, bbuf)
        spt = SLABP // NS
        iota = _iota16()

        def do_slab(si):
            slab_base = si * SLAB
            _zero_rows(zbuf, slab, tid * spt, spt)
            plsc.subcore_barrier()

            e0 = _vmem_scalar(bbuf, si)
            e1 = _vmem_scalar(bbuf, si + 1)
            per = (e1 - e0 + NS - 1) // NS
            my0 = e0 + tid * per
            my1 = jnp.minimum(my0 + per, e1)
            my0a = (my0 // 8) * 8
            nbatch = jnp.maximum((my1 - my0a + B - 1) // B, 0)

            def batch(bi, _):
                eb = my0a + bi * B
                pltpu.sync_copy(te0_hbm.at[pl.ds(eb, B)], sbuf)
                pltpu.sync_copy(te1_hbm.at[pl.ds(eb, B)], dbuf)
                pltpu.async_copy(h_hbm.at[sbuf], rowbuf, sem).wait()
                for j in range(B // L):
                    dd = dbuf[pl.ds(j * L, L)]
                    e_ids = eb + j * L + iota
                    valid = (e_ids >= my0) & (e_ids < my1)
                    lbuf[pl.ds(j * L, L)] = jnp.where(valid, dd - slab_base,
                                                      jnp.int32(SLAB))
                pltpu.sync_copy(rowbuf, slab.at[lbuf], add=True)
                return 0

            lax.fori_loop(0, nbatch, batch, 0)
            plsc.subcore_barrier()
            fpt = SLAB // NS
            pltpu.sync_copy(slab.at[pl.ds(tid * fpt, fpt)],
                            x4_hbm.at[pl.ds(slab_base + tid * fpt, fpt)])
            plsc.subcore_barrier()

        nmine = (n_slabs + 1) // 2

        def mine(i, _):
            si = i * 2 + cid

            @pl.when(si < n_slabs)
            def _():
                do_slab(si)
            return 0

        lax.fori_loop(0, nmine, mine, 0)

    return k


def _seg7_kernel(mp):
    half = mp // 2
    nb = half // (NS * B)
    tpr = ND // NS

    @functools.partial(
        pl.kernel,
        out_type=[
            jax.ShapeDtypeStruct((2, ND, D), jnp.float32),
            jax.ShapeDtypeStruct((ND, D), jnp.float32),
        ],
        mesh=_MESH,
        compiler_params=_SC_PARAMS,
        scratch_types=[
            pltpu.VMEM((B, D), jnp.float32),
            pltpu.VMEM((B,), jnp.int32),
            pltpu.VMEM_SHARED((ND, D), jnp.float32),
            pltpu.SemaphoreType.DMA,
        ],
    )
    def k(x4_hbm, node_hbm, dpos_hbm, s7_hbm, x4d_hbm, xbuf, ibuf, table, sem):
        cid = lax.axis_index("c")
        tid = lax.axis_index("s")
        _fill_zero(xbuf)
        _zero_rows(xbuf, table, tid * tpr, tpr)
        plsc.subcore_barrier()
        rpt = half // NS

        def body(b, _):
            base = cid * half + tid * rpt + b * B
            pltpu.sync_copy(node_hbm.at[pl.ds(base, B)], ibuf)
            pltpu.sync_copy(x4_hbm.at[pl.ds(base, B), :], xbuf)
            pltpu.sync_copy(xbuf, table.at[ibuf], add=True)
            return 0

        lax.fori_loop(0, nb, body, 0)
        plsc.subcore_barrier()
        pltpu.sync_copy(table.at[pl.ds(tid * tpr, tpr)],
                        s7_hbm.at[cid, pl.ds(tid * tpr, tpr), :])

        @pl.when(cid == 0)
        def _():
            def dbody(b, _):
                base = tid * tpr + b * B
                pltpu.sync_copy(dpos_hbm.at[pl.ds(base, B)], ibuf)
                pltpu.async_copy(x4_hbm.at[ibuf], xbuf, sem).wait()
                pltpu.sync_copy(xbuf, x4d_hbm.at[pl.ds(base, B), :])
                return 0
            lax.fori_loop(0, tpr // B, dbody, 0)

    return k


BG = 256


def _gsum_kernel(mp):
    rpt = mp // NS
    nb = rpt // BG
    tpr = ND // NS
    B2 = 128

    @functools.partial(
        pl.kernel,
        out_type=[
            jax.ShapeDtypeStruct((mp, D), jnp.float32),
            jax.ShapeDtypeStruct((mp, D), jnp.float32),
        ],
        mesh=_MESH,
        compiler_params=_SC_PARAMS,
        scratch_types=[
            pltpu.VMEM((BG, D), jnp.float32),
            pltpu.VMEM((B2, D), jnp.float32),
            pltpu.VMEM((BG,), jnp.int32),
            pltpu.VMEM((B2,), jnp.int32),
            pltpu.SemaphoreType.DMA,
        ],
    )
    def k(ts_hbm, tnp_hbm, tsd_hbm, subg_hbm, node_hbm, dpos_hbm,
          ga_hbm, gb_hbm, buf, buf2, ibuf, ibuf2, sem):
        cid = lax.axis_index("c")
        tid = lax.axis_index("s")

        @pl.when(cid == 0)
        def _():
            def body(b, _):
                base = tid * rpt + b * BG
                pltpu.sync_copy(subg_hbm.at[pl.ds(base, BG)], ibuf)
                pltpu.async_copy(ts_hbm.at[ibuf], buf, sem).wait()
                pltpu.sync_copy(buf, ga_hbm.at[pl.ds(base, BG), :])
                return 0
            lax.fori_loop(0, nb, body, 0)

        @pl.when(cid == 1)
        def _():
            def body(b, _):
                base = tid * rpt + b * BG
                pltpu.sync_copy(node_hbm.at[pl.ds(base, BG)], ibuf)
                pltpu.async_copy(tnp_hbm.at[ibuf], buf, sem).wait()
                pltpu.sync_copy(buf, gb_hbm.at[pl.ds(base, BG), :])
                return 0
            lax.fori_loop(0, nb, body, 0)

        plsc.subcore_barrier()

        @pl.when(cid == 0)
        def _():
            def dbody(j, _):
                base = tid * tpr + j * B2
                pltpu.sync_copy(dpos_hbm.at[pl.ds(base, B2)], ibuf2)
                pltpu.sync_copy(tsd_hbm.at[pl.ds(base, B2), :], buf2)
                pltpu.sync_copy(buf2, ga_hbm.at[ibuf2])
                return 0
            lax.fori_loop(0, tpr // B2, dbody, 0)

    return k


def kernel(X, W_lin0, b_lin0, W_het, W_mlp1, b_mlp1, subg_idx, node_idx, tuple_edge):
    m = X.shape[0]
    k_edges = tuple_edge.shape[1]

    chunk = 2 * SLAB
    mp = ((m + chunk - 1) // chunk) * chunk

    subg = subg_idx.astype(jnp.int32)
    node = node_idx.astype(jnp.int32)
    te0 = tuple_edge[0].astype(jnp.int32)
    te1 = tuple_edge[1].astype(jnp.int32)

    xp = jnp.pad(X, ((0, mp - m), (0, 0)))
    subg_p = jnp.pad(subg, (0, mp - m), constant_values=N)
    node_p = jnp.pad(node, (0, mp - m), constant_values=N)
    dsel_p = jnp.pad(jnp.where(subg == node, subg, N), (0, mp - m), constant_values=N)
    kchunk = NC * NS * B
    kp = ((k_edges + 2 * B + kchunk - 1) // kchunk) * kchunk
    te0_p = jnp.pad(te0, (0, kp - k_edges), constant_values=0)
    te1_p = jnp.pad(te1, (0, kp - k_edges), constant_values=0)

    codes = subg * N + node
    dpos = jnp.searchsorted(codes, jnp.arange(N, dtype=jnp.int32) * (N + 1)).astype(jnp.int32)
    dpos_p = jnp.pad(dpos, (0, ND - N), constant_values=m)

    n_slabs = (mp + SLAB - 1) // SLAB
    nbounds = ((n_slabs + 1 + L - 1) // L) * L
    slab_bounds = jnp.searchsorted(te1, jnp.arange(n_slabs + 1, dtype=jnp.int32) * SLAB).astype(jnp.int32)
    slab_bounds = jnp.pad(slab_bounds, (0, nbounds - n_slabs - 1), constant_values=k_edges)

    bnd = jnp.arange(N + 1, dtype=jnp.int32)
    cnt_s1 = jnp.diff(jnp.searchsorted(subg, bnd)).astype(jnp.float32)
    cnt_n1 = jnp.diff(jnp.searchsorted(jnp.sort(node), bnd)).astype(jnp.float32)
    rcp_s = jnp.broadcast_to(jnp.pad(1.0 / jnp.maximum(cnt_s1, 1.0), (0, ND - N))[:, None], (ND, D))
    rcp_n = jnp.broadcast_to(jnp.pad(1.0 / jnp.maximum(cnt_n1, 1.0), (0, ND - N))[:, None], (ND, D))

    call = _fold_weights(W_het.reshape(2 * 7 * D, D), W_mlp1).reshape(2, 7, D, D)
    c1 = call[0, 0]
    c4 = call[0, 3]
    b0 = b_lin0.reshape(1, D)
    bm1 = b_mlp1.reshape(1, D)

    h, g1 = _pass1(xp, W_lin0, b0, c1, 512)

    s_s, s_n, dvec = _seg1_kernel(mp)(xp, subg_p, node_p, dpos_p)

    x4 = _x4_kernel(mp, n_slabs, nbounds)(h, te0_p, te1_p, slab_bounds)

    s7p, x4d = _seg7_kernel(mp)(x4, node_p, dpos_p)

    ts, tnp, tsd = _tables(s_s, s_n, s7p, dvec, x4d, rcp_s, rcp_n, call)

    ga, gb = _gsum_kernel(mp)(ts, tnp, tsd, subg_p, node_p, dpos_p)

    out = _final(x4, g1, ga, gb, c4, bm1, 512)
    return out[:m]
